# Initial kernel scaffold; baseline (speedup 1.0000x reference)
import jax
import jax.numpy as jnp
from jax.experimental import pallas as pl


def kernel(x_scalar, x_rot, edge_index, distance_embedding, rot, W1, b1, W2, b2):
    raise NotImplementedError("write your pallas kernel here")


# speedup vs baseline: 27.8273x; 27.8273x over previous
import functools

import jax
import jax.numpy as jnp
import numpy as np
from jax import lax
from jax.experimental import pallas as pl
from jax.experimental.pallas import tpu as pltpu
from jax.experimental.pallas import tpu_sc as plsc

N_SCALARS = 8
NUM_REP = 4
L_MAX = 2
X_DIM = 24
DIST = 16
HID = 72
PADW = 32

_MESH = dict(core_axis_name="c", subcore_axis_name="s", num_cores=2,
             num_subcores=16)
NW = 32
IDXB = 80
CH = 800
NB = CH // IDXB


def _sc_gather(table, idx2d, n_edges):
    per_w = n_edges // NW
    n_chunks = per_w // CH

    @functools.partial(
        pl.kernel,
        mesh=plsc.VectorSubcoreMesh(**_MESH),
        compiler_params=pltpu.CompilerParams(use_tc_tiling_on_sc=False),
        out_type=jax.ShapeDtypeStruct((n_edges, PADW), jnp.float32),
        scratch_types=[
            pltpu.VMEM((CH,), jnp.int32),
            pltpu.VMEM((CH, PADW), jnp.float32),
            pltpu.SemaphoreType.DMA,
        ],
    )
    def gk(table_hbm, idx_hbm, out_hbm, idx_v, rows_v, sem):
        wid = lax.axis_index("s") * 2 + lax.axis_index("c")
        base = wid * per_w

        def body(i, carry):
            start = pl.multiple_of(base + i * CH, CH)
            pltpu.sync_copy(idx_hbm.at[pl.ds(start, CH)], idx_v)
            handles = []
            for b in range(NB):
                handles.append(pltpu.async_copy(
                    table_hbm.at[idx_v.at[pl.ds(b * IDXB, IDXB)]],
                    rows_v.at[pl.ds(b * IDXB, IDXB)], sem))
            for h in handles:
                h.wait()
            pltpu.sync_copy(rows_v, out_hbm.at[pl.ds(start, CH)])
            return carry

        lax.fori_loop(0, n_chunks, body, 0)

    return gk(table, idx2d)


def _dense_body(g_ref, de_ref, rot_ref, w1_ref, b1_ref, w2_ref, b2_ref,
                out_ref):
    g = g_ref[...]
    de = de_ref[...]
    rb = rot_ref[...]
    bsz = g.shape[0]
    lane8 = lax.broadcasted_iota(jnp.int32, (bsz, 8), 1)

    def coef(c_lo, c_hi):
        r0 = rb[:, c_lo:c_lo + 1]
        r1 = rb[:, c_hi:c_hi + 1]
        return jnp.where(lane8 < 4, r0, r1)

    s = g[:, 0:8]
    x0 = g[:, 8:16]
    x1 = g[:, 16:24]
    y0 = x0 * coef(0, 4) + x1 * coef(1, 5)
    y1 = x0 * coef(2, 6) + x1 * coef(3, 7)

    f32 = jnp.float32
    h = (jnp.dot(de, w1_ref[0:16, :], preferred_element_type=f32)
         + jnp.dot(s, w1_ref[16:24, :], preferred_element_type=f32)
         + jnp.dot(y0, w1_ref[24:32, :], preferred_element_type=f32)
         + jnp.dot(y1, w1_ref[32:40, :], preferred_element_type=f32)
         + b1_ref[...])
    h = jnp.maximum(h, 0.0)
    o = jnp.dot(h, w2_ref[...], preferred_element_type=f32) + b2_ref[...]

    os_ = o[:, 0:8]
    m0 = o[:, 8:16]
    m1 = o[:, 16:24]
    z0 = m0 * coef(0, 4) + m1 * coef(2, 6)
    z1 = m0 * coef(1, 5) + m1 * coef(3, 7)
    out_ref[...] = jnp.concatenate(
        [os_, z0, z1, jnp.zeros_like(z0)], axis=1)


def _dense_stage(gathered, de, rot_flat, w1p, b1, w2p, b2p, blk=2000):
    n_edges = gathered.shape[0]
    return pl.pallas_call(
        _dense_body,
        grid=(n_edges // blk,),
        in_specs=[
            pl.BlockSpec((blk, PADW), lambda i: (i, 0)),
            pl.BlockSpec((blk, DIST), lambda i: (i, 0)),
            pl.BlockSpec((blk, 8), lambda i: (i, 0)),
            pl.BlockSpec((40, HID), lambda i: (0, 0)),
            pl.BlockSpec((1, HID), lambda i: (0, 0)),
            pl.BlockSpec((HID, PADW), lambda i: (0, 0)),
            pl.BlockSpec((1, PADW), lambda i: (0, 0)),
        ],
        out_specs=pl.BlockSpec((blk, PADW), lambda i: (i, 0)),
        out_shape=jax.ShapeDtypeStruct((n_edges, PADW), jnp.float32),
    )(gathered, de, rot_flat, w1p, b1, w2p, b2p)


def _sc_scatter(msgs, col2d, n_edges, n_nodes):
    half = n_nodes // 2
    acc_rows = 51200
    per_tec = n_edges // 16
    n_chunks = per_tec // CH
    zero_rows_per_tec = acc_rows // 16
    drain_a = 3128
    drain_b = half - 15 * drain_a

    @functools.partial(
        pl.kernel,
        mesh=plsc.VectorSubcoreMesh(**_MESH),
        compiler_params=pltpu.CompilerParams(use_tc_tiling_on_sc=False),
        out_type=jax.ShapeDtypeStruct((n_nodes, PADW), jnp.float32),
        scratch_types=[
            pltpu.VMEM((CH,), jnp.int32),
            pltpu.VMEM((NB, IDXB), jnp.int32),
            pltpu.VMEM((CH, PADW), jnp.float32),
            pltpu.VMEM_SHARED((acc_rows, PADW), jnp.float32),
            pltpu.SemaphoreType.DMA,
        ],
    )
    def sk(msg_hbm, col_hbm, out_hbm, col_v, loc_v, msg_v, acc, sem):
        c = lax.axis_index("c")
        t = lax.axis_index("s")
        node_base = c * half

        zeros16 = jnp.zeros((16,), jnp.float32)

        def zbody(i, carry):
            msg_v[i // 2, pl.ds((i % 2) * 16 + 0, 16)] = zeros16
            return carry
        lax.fori_loop(0, CH * 2, zbody, 0)
        for q in range(zero_rows_per_tec // CH):
            pltpu.sync_copy(
                msg_v, acc.at[pl.ds(t * zero_rows_per_tec + q * CH, CH)])
        plsc.subcore_barrier()

        def body(i, carry):
            start = pl.multiple_of(t * per_tec + i * CH, CH)
            pltpu.sync_copy(col_hbm.at[pl.ds(start, CH)], col_v)
            pltpu.sync_copy(msg_hbm.at[pl.ds(start, CH)], msg_v)
            for b in range(NB):
                for gsub in range(IDXB // 16):
                    v = col_v[pl.ds(b * IDXB + gsub * 16, 16)]
                    li = v - node_base
                    ok = (li >= 0) & (li < half)
                    loc_v[b, pl.ds(gsub * 16, 16)] = jnp.where(ok, li, half)
            handles = []
            for b in range(NB):
                handles.append(pltpu.async_copy(
                    msg_v.at[pl.ds(b * IDXB, IDXB)],
                    acc.at[loc_v.at[b]], sem, add=True))
            for h in handles:
                h.wait()
            return carry

        lax.fori_loop(0, n_chunks, body, 0)
        plsc.subcore_barrier()
        off = pl.multiple_of(t * drain_a, 8)
        @pl.when(t < 15)
        def _():
            pltpu.sync_copy(
                acc.at[pl.ds(off, drain_a)],
                out_hbm.at[pl.ds(pl.multiple_of(node_base + off, 8),
                                 drain_a)])
        @pl.when(t == 15)
        def _():
            pltpu.sync_copy(
                acc.at[pl.ds(15 * drain_a, drain_b)],
                out_hbm.at[pl.ds(pl.multiple_of(node_base + 15 * drain_a, 8),
                                 drain_b)])

    return sk(msgs, col2d)


def kernel(x_scalar, x_rot, edge_index, distance_embedding, rot, W1, b1, W2,
           b2):
    n_nodes = x_rot.shape[0]
    n_edges = edge_index.shape[1]

    row = edge_index[0].astype(jnp.int32)
    col = edge_index[1].astype(jnp.int32)

    xr = x_rot.reshape(n_nodes, NUM_REP, L_MAX, 2)
    xr_m = jnp.transpose(xr, (0, 3, 2, 1)).reshape(n_nodes, 16)
    table = jnp.concatenate(
        [x_scalar, xr_m, jnp.zeros((n_nodes, 8), jnp.float32)], axis=1)

    perm_in = list(range(24)) + [
        24 + j * 4 + k * 2 + l
        for l in range(2) for k in range(2) for j in range(4)]
    w1p = W1[np.array(perm_in), :]
    perm_out = list(range(8)) + [
        8 + j * 4 + k * 2 + m
        for m in range(2) for k in range(2) for j in range(4)]
    w2p = jnp.concatenate(
        [W2[:, np.array(perm_out)], jnp.zeros((HID, 8), jnp.float32)], axis=1)
    b2p = jnp.concatenate(
        [b2[np.array(perm_out)], jnp.zeros((8,), jnp.float32)])

    gathered = _sc_gather(table, row, n_edges)
    msgs = _dense_stage(gathered, distance_embedding,
                        rot.reshape(n_edges, 8), w1p, b1.reshape(1, HID),
                        w2p, b2p.reshape(1, PADW))
    acc = _sc_scatter(msgs, col, n_edges, n_nodes)

    mess_scalar = acc[:, :N_SCALARS]
    zm = acc[:, 8:24].reshape(n_nodes, 2, 2, NUM_REP)
    mess_rot = jnp.transpose(zm, (0, 3, 2, 1)).reshape(
        n_nodes, NUM_REP, L_MAX * 2)
    return (mess_scalar, mess_rot)
